# Initial kernel scaffold; baseline (speedup 1.0000x reference)
import jax
import jax.numpy as jnp
from jax.experimental import pallas as pl


def kernel(x, edge_index, W1, b1, W2, b2):
    raise NotImplementedError("write your pallas kernel here")


# speedup vs baseline: 26.6764x; 26.6764x over previous
import functools

import jax
import jax.numpy as jnp
from jax import lax
from jax.experimental import pallas as pl
from jax.experimental.pallas import tpu as pltpu
from jax.experimental.pallas import tpu_sc as plsc

N = 10000
E = 320000
IN_C = 128
HID = 64
OUT_C = 128

NC = 2
NS = 16
NW = NC * NS
EPW = E // NW
CH = 80
NCHUNK = EPW // CH
NPAD = 10240
SPT = NPAD // NS

RB = 2048
GRID = -(-N // RB)

def _mesh():
    return plsc.VectorSubcoreMesh(
        core_axis_name="c", subcore_axis_name="s", num_cores=NC, num_subcores=NS
    )


@functools.cache
def _make_sc_hist():
    @functools.partial(
        pl.kernel,
        out_type=jax.ShapeDtypeStruct((NW, 1, N), jnp.float32),
        mesh=_mesh(),
        compiler_params=pltpu.CompilerParams(
            needs_layout_passes=False, use_tc_tiling_on_sc=False
        ),
        scratch_types=[
            pltpu.VMEM((NCHUNK, CH), jnp.int32),
            pltpu.VMEM((N,), jnp.float32),
        ],
    )
    def _sc_hist(e2d_hbm, zeros1_hbm, out_hbm, dstbuf, hist):
        w = lax.axis_index("s") * NC + lax.axis_index("c")
        pltpu.sync_copy(zeros1_hbm, hist)
        pltpu.sync_copy(e2d_hbm.at[1, w], dstbuf)
        ones = jnp.full((16,), 1.0, jnp.float32)

        def body(j, carry):
            for k in range(CH // 16):
                d = dstbuf[j, pl.ds(k * 16, 16)]
                plsc.addupdate_scatter(hist, [d], ones)
            return carry

        lax.fori_loop(0, NCHUNK, body, 0)
        pltpu.sync_copy(hist, out_hbm.at[w, 0])

    return _sc_hist


@functools.cache
def _make_sc_agg():
    @functools.partial(
        pl.kernel,
        out_type=jax.ShapeDtypeStruct((NC, NPAD, HID), jnp.float32),
        mesh=_mesh(),
        compiler_params=pltpu.CompilerParams(
            needs_layout_passes=False, use_tc_tiling_on_sc=False
        ),
        scratch_types=[
            pltpu.VMEM((NCHUNK, CH), jnp.int32),
            pltpu.VMEM((NCHUNK, CH), jnp.int32),
            pltpu.VMEM((CH, HID), jnp.float32),
            pltpu.VMEM_SHARED((NPAD, HID), jnp.float32),
            pltpu.SemaphoreType.DMA,
        ],
    )
    def _sc_agg(hp_hbm, e2d_hbm, zeros2_hbm, out_hbm, src_idx, dst_idx, rows, acc, sem):
        c = lax.axis_index("c")
        s = lax.axis_index("s")
        w = s * NC + c
        pltpu.sync_copy(
            zeros2_hbm.at[pl.ds(s * SPT, SPT)],
            acc.at[pl.ds(s * SPT, SPT)],
        )
        pltpu.sync_copy(e2d_hbm.at[0, w], src_idx)
        pltpu.sync_copy(e2d_hbm.at[1, w], dst_idx)
        plsc.subcore_barrier()

        def body(j, carry):
            pltpu.async_copy(hp_hbm.at[src_idx.at[j]], rows, sem).wait()
            pltpu.sync_copy(rows, acc.at[dst_idx.at[j]], add=True)
            return carry

        lax.fori_loop(0, NCHUNK, body, 0)
        plsc.subcore_barrier()
        pltpu.sync_copy(
            acc.at[pl.ds(s * SPT, SPT)],
            out_hbm.at[c, pl.ds(s * SPT, SPT)],
        )

    return _sc_agg


def _tc1_body(x_ref, w1_ref, hist_ref, h1p_ref, dis_ref):
    deg = 1.0 + jnp.sum(hist_ref[:, 0, :], axis=0)
    dis = lax.rsqrt(deg)[:, None]
    h1 = jnp.dot(x_ref[...], w1_ref[...], preferred_element_type=jnp.float32)
    h1p_ref[...] = h1 * dis
    dis_ref[...] = dis


def _tc1(x, W1, hist):
    return pl.pallas_call(
        _tc1_body,
        grid=(GRID,),
        in_specs=[
            pl.BlockSpec((RB, IN_C), lambda i: (i, 0)),
            pl.BlockSpec((IN_C, HID), lambda i: (0, 0)),
            pl.BlockSpec((NW, 1, RB), lambda i: (0, 0, i)),
        ],
        out_specs=[
            pl.BlockSpec((RB, HID), lambda i: (i, 0)),
            pl.BlockSpec((RB, 1), lambda i: (i, 0)),
        ],
        out_shape=[
            jax.ShapeDtypeStruct((N, HID), jnp.float32),
            jax.ShapeDtypeStruct((N, 1), jnp.float32),
        ],
    )(x, W1, hist)


def _tc2_body(p_ref, h1p_ref, dis_ref, b1_ref, hp_ref):
    t = p_ref[0] + p_ref[1] + h1p_ref[...]
    h = jnp.maximum(dis_ref[...] * t + b1_ref[...], 0.0)
    hp_ref[...] = h * dis_ref[...]


def _tc2(p, h1p, dis, b1):
    return pl.pallas_call(
        _tc2_body,
        grid=(GRID,),
        in_specs=[
            pl.BlockSpec((NC, RB, HID), lambda i: (0, i, 0)),
            pl.BlockSpec((RB, HID), lambda i: (i, 0)),
            pl.BlockSpec((RB, 1), lambda i: (i, 0)),
            pl.BlockSpec((1, HID), lambda i: (0, 0)),
        ],
        out_specs=pl.BlockSpec((RB, HID), lambda i: (i, 0)),
        out_shape=jax.ShapeDtypeStruct((N, HID), jnp.float32),
    )(p, h1p, dis, b1)


def _tc3_body(p_ref, hp_ref, dis_ref, w2_ref, b2_ref, out_ref):
    a2 = dis_ref[...] * (p_ref[0] + p_ref[1] + hp_ref[...])
    z = jnp.dot(a2, w2_ref[...], preferred_element_type=jnp.float32) + b2_ref[...]
    m = jnp.max(z, axis=1, keepdims=True)
    lse = jnp.log(jnp.sum(jnp.exp(z - m), axis=1, keepdims=True)) + m
    out_ref[...] = z - lse


def _tc3(p, hp, dis, W2, b2):
    return pl.pallas_call(
        _tc3_body,
        grid=(GRID,),
        in_specs=[
            pl.BlockSpec((NC, RB, HID), lambda i: (0, i, 0)),
            pl.BlockSpec((RB, HID), lambda i: (i, 0)),
            pl.BlockSpec((RB, 1), lambda i: (i, 0)),
            pl.BlockSpec((HID, OUT_C), lambda i: (0, 0)),
            pl.BlockSpec((1, OUT_C), lambda i: (0, 0)),
        ],
        out_specs=pl.BlockSpec((RB, OUT_C), lambda i: (i, 0)),
        out_shape=jax.ShapeDtypeStruct((N, OUT_C), jnp.float32),
    )(p, hp, dis, W2, b2)


def kernel(x, edge_index, W1, b1, W2, b2):
    e2d = edge_index.reshape(2, NW, NCHUNK, CH)
    zeros1 = jnp.zeros((N,), jnp.float32)
    zeros2 = jnp.zeros((NPAD, HID), jnp.float32)
    sc_hist, sc_agg = _make_sc_hist(), _make_sc_agg()
    hist = sc_hist(e2d, zeros1)
    h1p, dis = _tc1(x, W1, hist)
    p1 = sc_agg(h1p, e2d, zeros2)
    hp = _tc2(p1, h1p, dis, b1.reshape(1, HID))
    p2 = sc_agg(hp, e2d, zeros2)
    return _tc3(p2, hp, dis, W2, b2.reshape(1, OUT_C))
